# Initial kernel scaffold; baseline (speedup 1.0000x reference)
import jax
import jax.numpy as jnp
from jax.experimental import pallas as pl


def kernel(x, memory, params):
    raise NotImplementedError("write your pallas kernel here")


# speedup vs baseline: 1.4544x; 1.4544x over previous
import functools

import jax
import jax.numpy as jnp
from jax import lax
from jax.experimental import pallas as pl
from jax.experimental.pallas import tpu as pltpu
from jax.experimental.pallas import tpu_sc as plsc

B = 2
S = 2048
D_MODEL = 1024
D_CACHE = 64
N_SLOTS = 4096
D_TEMP = 16
D_SLOT = D_CACHE + 1 + D_TEMP

TS = 256
NT = S // TS

SV_PAD = 96

_RGB, _WGB, _RSC, _WSC, _TMP, _RLD, _RGB2, _IMB2, _WDB2 = range(9)


def _ln(x, g, b):
    m = x.mean(-1, keepdims=True)
    v = ((x - m) ** 2).mean(-1, keepdims=True)
    return (x - m) / jnp.sqrt(v + 1e-5) * g + b


def _tc_body(scal_ref, x_ref, content_ref, valid_ref,
             wq_ref, bq_ref, wto_ref, bto_ref, wfrom_ref, bfrom_ref,
             wf1_ref, wf2_ref, bfuse_ref,
             rg_lng_ref, rg_lnb_ref, rg_w1_ref, rg_b1_ref, rg_w2_ref,
             im_lng_ref, im_lnb_ref, im_w1_ref, im_b1_ref, im_w2_ref,
             wd_lng_ref, wd_lnb_ref, wd_w1_ref, wd_b1_ref, wd_w2_ref,
             temporal_ref,
             out_ref, v2_ref, slotg_ref, slotvec_ref,
             smax_ref, best_ref, bestvec_ref, wm_ref):
    b = pl.program_id(0)
    t = pl.program_id(1)

    x = x_ref[0]
    content = content_ref[0]
    valid = valid_ref[0]

    q = jnp.dot(x, wq_ref[...]) + bq_ref[...]
    scores = lax.dot_general(q, content,
                             (((1,), (1,)), ((), ()))) / jnp.sqrt(float(D_CACHE))
    scores = jnp.where(valid > 0.5, scores, -1e9)
    m = jnp.max(scores, axis=-1, keepdims=True)
    e = jnp.exp(scores - m)
    attn = e / jnp.sum(e, axis=-1, keepdims=True)
    read = jnp.dot(attn, content)
    context = jnp.dot(read, wfrom_ref[...]) + bfrom_ref[...]

    gl = jnp.dot(jax.nn.silu(
        jnp.dot(_ln(x, rg_lng_ref[...], rg_lnb_ref[...]), rg_w1_ref[...])
        + rg_b1_ref[...]), rg_w2_ref[...])[:, :1] + scal_ref[_RGB2]
    r = jax.nn.sigmoid(gl + scal_ref[_RGB])
    base = jnp.clip(jax.nn.sigmoid(scal_ref[_RLD]), 0.1, 0.99)
    expo = jnp.clip(8.0 * r * jnp.log(base), -20.0, 0.0)
    read_gate = 1.0 - jnp.exp(expo)

    fused = jnp.dot(x, wf1_ref[...]) + jnp.dot(context, wf2_ref[...]) + bfuse_ref[...]
    out_ref[0] = x + read_gate * fused * scal_ref[_RSC]

    pmax = jnp.max(attn, axis=0, keepdims=True)
    @pl.when(t == 0)
    def _():
        smax_ref[...] = pmax
        best_ref[0] = -1.0
    @pl.when(t != 0)
    def _():
        smax_ref[...] = jnp.maximum(smax_ref[...], pmax)

    wm = jnp.dot(x, wto_ref[...]) + bto_ref[...]
    wm_ref[...] = wm
    il = jnp.dot(jax.nn.silu(
        jnp.dot(_ln(x, im_lng_ref[...], im_lnb_ref[...]), im_w1_ref[...])
        + im_b1_ref[...]), im_w2_ref[...])[:, :1] + scal_ref[_IMB2]
    imp = jax.nn.sigmoid(il) * jnp.abs(scal_ref[_WSC])
    cat = jnp.concatenate([q, read], axis=-1)
    wl = jnp.dot(jax.nn.silu(
        jnp.dot(_ln(cat, wd_lng_ref[...], wd_lnb_ref[...]), wd_w1_ref[...])
        + wd_b1_ref[...]), wd_w2_ref[...])[:, :1] + scal_ref[_WDB2]
    temp = jnp.maximum(scal_ref[_TMP], 0.1)
    adj = (wl + scal_ref[_WGB]) / temp
    ew = jnp.exp(jnp.minimum(adj, 10.0))
    strength = imp * (ew / (1.0 + ew))

    mt = jnp.max(strength)
    ridx = lax.broadcasted_iota(jnp.int32, (TS, 1), 0)
    it = jnp.min(jnp.where(strength >= mt, ridx, TS))

    @pl.when(mt > best_ref[0])
    def _():
        best_ref[0] = mt
        bestvec_ref[...] = wm_ref[pl.ds(it, 1), :]

    @pl.when(t == NT - 1)
    def _():
        v2 = valid * (1.0 - smax_ref[...])
        v2_ref[0] = v2
        mn = jnp.min(v2)
        cidx = lax.broadcasted_iota(jnp.int32, (1, N_SLOTS), 1)
        slot = jnp.min(jnp.where(v2 <= mn, cidx, N_SLOTS))
        slotg_ref[0] = jnp.broadcast_to(b * N_SLOTS + slot, (1, 1))
        bs = jnp.broadcast_to(jnp.clip(best_ref[0], 0.0, 1.0), (1, 1))
        slotvec_ref[0] = jnp.concatenate(
            [bestvec_ref[...], bs, temporal_ref[...],
             jnp.zeros((1, SV_PAD - D_SLOT), jnp.float32)], axis=-1)


def _tc_call(x, content, valid_row, scal, weights):
    (wq, bq, wto, bto, wfrom, bfrom, wf1, wf2, bfuse,
     rg_lng, rg_lnb, rg_w1, rg_b1, rg_w2,
     im_lng, im_lnb, im_w1, im_b1, im_w2,
     wd_lng, wd_lnb, wd_w1, wd_b1, wd_w2, temporal) = weights

    full = lambda shape: pl.BlockSpec(shape, lambda b, t: (0,) * len(shape))
    grid = (B, NT)
    specs = [
        pl.BlockSpec(memory_space=pltpu.SMEM),
        pl.BlockSpec((1, TS, D_MODEL), lambda b, t: (b, t, 0)),
        pl.BlockSpec((1, N_SLOTS, D_CACHE), lambda b, t: (b, 0, 0)),
        pl.BlockSpec((1, 1, N_SLOTS), lambda b, t: (b, 0, 0)),
    ] + [full(w.shape) for w in weights]
    out_shapes = [
        jax.ShapeDtypeStruct((B, S, D_MODEL), jnp.float32),
        jax.ShapeDtypeStruct((B, 1, N_SLOTS), jnp.float32),
        jax.ShapeDtypeStruct((B, 1, 1), jnp.int32),
        jax.ShapeDtypeStruct((B, 1, SV_PAD), jnp.float32),
    ]
    out_specs = [
        pl.BlockSpec((1, TS, D_MODEL), lambda b, t: (b, t, 0)),
        pl.BlockSpec((1, 1, N_SLOTS), lambda b, t: (b, 0, 0)),
        pl.BlockSpec((1, 1, 1), lambda b, t: (b, 0, 0)),
        pl.BlockSpec((1, 1, SV_PAD), lambda b, t: (b, 0, 0)),
    ]
    return pl.pallas_call(
        _tc_body,
        grid=grid,
        in_specs=specs,
        out_specs=out_specs,
        out_shape=out_shapes,
        scratch_shapes=[
            pltpu.VMEM((1, N_SLOTS), jnp.float32),
            pltpu.SMEM((1,), jnp.float32),
            pltpu.VMEM((1, D_CACHE), jnp.float32),
            pltpu.VMEM((TS, D_CACHE), jnp.float32),
        ],
        compiler_params=pltpu.CompilerParams(
            dimension_semantics=("arbitrary", "arbitrary")),
    )(scal, x, content, valid_row, *weights)


ROWS_PER_W = B * N_SLOTS // 32
WORDS_PER_W = ROWS_PER_W * D_SLOT


def _sc_body(mem_hbm, v2_hbm, slotg_hbm, slotvec_hbm, out_hbm,
             buf, v2v, svv, sgv):
    wid = lax.axis_index("s") * 2 + lax.axis_index("c")
    base_r = wid * ROWS_PER_W
    base_w = wid * WORDS_PER_W
    pltpu.sync_copy(mem_hbm.at[pl.ds(base_w, WORDS_PER_W)], buf)
    pltpu.sync_copy(v2_hbm.at[pl.ds(base_r, ROWS_PER_W)], v2v)
    pltpu.sync_copy(slotg_hbm, sgv)
    pltpu.sync_copy(slotvec_hbm, svv)
    lanes = lax.iota(jnp.int32, 16)
    for i in range(ROWS_PER_W // 16):
        idx = (i * 16 + lanes) * D_SLOT + D_CACHE
        plsc.store_scatter(buf, [idx], v2v[pl.ds(i * 16, 16)])
    sg = sgv[...]
    for bb in range(B):
        g = jnp.max(jnp.where(lanes == bb, sg, -1))
        local = g - base_r
        @pl.when((local >= 0) & (local < ROWS_PER_W))
        def _():
            for c in range(SV_PAD // 16):
                widx = local * D_SLOT + c * 16 + lanes
                mask = (c * 16 + lanes) < D_SLOT
                plsc.store_scatter(buf, [widx],
                                   svv[pl.ds(bb * SV_PAD + c * 16, 16)],
                                   mask=mask)
    pltpu.sync_copy(buf, out_hbm.at[pl.ds(base_w, WORDS_PER_W)])


@functools.cache
def _sc_update_kernel():
    return functools.partial(
        pl.kernel,
        mesh=plsc.VectorSubcoreMesh(core_axis_name="c", subcore_axis_name="s",
                                    num_cores=2),
        out_type=jax.ShapeDtypeStruct((B * N_SLOTS * D_SLOT,), jnp.float32),
        scratch_types=[
            pltpu.VMEM((WORDS_PER_W,), jnp.float32),
            pltpu.VMEM((ROWS_PER_W,), jnp.float32),
            pltpu.VMEM((B * SV_PAD,), jnp.float32),
            pltpu.VMEM((16,), jnp.int32),
        ],
        compiler_params=pltpu.CompilerParams(use_tc_tiling_on_sc=False,
                                             needs_layout_passes=False),
    )(_sc_body)


def kernel(x, memory, params):
    p = params
    content = memory[..., :D_CACHE]
    valid_row = memory[:, :, D_CACHE][:, None, :]

    scal = jnp.stack([
        p['read_gate_bias'], p['write_gate_bias'], p['read_scale'],
        p['write_scale'], p['write_temperature'], p['read_log_decay'],
        p['rg_b2'][0], p['im_b2'][0], p['wd_b2'][0],
    ]).astype(jnp.float32)

    row = lambda a: a.reshape(1, -1)
    weights = (
        p['Wq'], row(p['bq']), p['Wto'], row(p['bto']),
        p['Wfrom'], row(p['bfrom']),
        p['Wfuse'][:D_MODEL], p['Wfuse'][D_MODEL:], row(p['bfuse']),
        row(p['rg_lng']), row(p['rg_lnb']), p['rg_w1'], row(p['rg_b1']),
        p['rg_w2'],
        row(p['im_lng']), row(p['im_lnb']), p['im_w1'], row(p['im_b1']),
        p['im_w2'],
        row(p['wd_lng']), row(p['wd_lnb']), p['wd_w1'], row(p['wd_b1']),
        p['wd_w2'],
        row(p['temporal_vec']),
    )

    out, v2, slotg, slotvec = _tc_call(x, content, valid_row, scal, weights)

    sg16 = jnp.zeros((16,), jnp.int32).at[:B].set(slotg.reshape(B))
    new_mem = _sc_update_kernel()(memory.reshape(-1), v2.reshape(-1), sg16,
                                  slotvec.reshape(-1))
    return out, new_mem.reshape(B, N_SLOTS, D_SLOT)


# speedup vs baseline: 1.5276x; 1.0503x over previous
import functools

import jax
import jax.numpy as jnp
from jax import lax
from jax.experimental import pallas as pl
from jax.experimental.pallas import tpu as pltpu
from jax.experimental.pallas import tpu_sc as plsc

B = 2
S = 2048
D_MODEL = 1024
D_CACHE = 64
N_SLOTS = 4096
D_TEMP = 16
D_SLOT = D_CACHE + 1 + D_TEMP

TS = 256
NT = S // TS

SV_PAD = 96

_RGB, _WGB, _RSC, _WSC, _TMP, _RLD, _RGB2, _IMB2, _WDB2 = range(9)


def _ln(x, g, b):
    m = x.mean(-1, keepdims=True)
    v = ((x - m) ** 2).mean(-1, keepdims=True)
    return (x - m) / jnp.sqrt(v + 1e-5) * g + b


def _tc_body(scal_ref, x_ref, content_ref, valid_ref,
             wq_ref, bq_ref, wto_ref, bto_ref, wfrom_ref, bfrom_ref,
             wf1_ref, wf2_ref, bfuse_ref,
             rg_lng_ref, rg_lnb_ref, rg_w1_ref, rg_b1_ref, rg_w2_ref,
             im_lng_ref, im_lnb_ref, im_w1_ref, im_b1_ref, im_w2_ref,
             wd_lng_ref, wd_lnb_ref, wd_w1_ref, wd_b1_ref, wd_w2_ref,
             temporal_ref,
             out_ref, v2_ref, slotg_ref, slotvec_ref,
             smax_ref, best_ref, bestvec_ref, wm_ref):
    b = pl.program_id(0)
    t = pl.program_id(1)

    x = x_ref[0]
    content = content_ref[0]
    valid = valid_ref[0]

    q = jnp.dot(x, wq_ref[...]) + bq_ref[...]
    scores = lax.dot_general(q, content,
                             (((1,), (1,)), ((), ()))) / jnp.sqrt(float(D_CACHE))
    scores = jnp.where(valid > 0.5, scores, -1e9)
    e = jnp.exp(scores)
    attn = e * (1.0 / jnp.sum(e, axis=-1, keepdims=True))
    read = jnp.dot(attn, content)
    context = jnp.dot(read, wfrom_ref[...]) + bfrom_ref[...]

    gl = jnp.dot(jax.nn.silu(
        jnp.dot(_ln(x, rg_lng_ref[...], rg_lnb_ref[...]).astype(jnp.bfloat16),
                rg_w1_ref[...], preferred_element_type=jnp.float32)
        + rg_b1_ref[...]), rg_w2_ref[...])[:, :1] + scal_ref[_RGB2]
    r = jax.nn.sigmoid(gl + scal_ref[_RGB])
    base = jnp.clip(jax.nn.sigmoid(scal_ref[_RLD]), 0.1, 0.99)
    expo = jnp.clip(8.0 * r * jnp.log(base), -20.0, 0.0)
    read_gate = 1.0 - jnp.exp(expo)

    fused = (jnp.dot(x.astype(jnp.bfloat16), wf1_ref[...],
                     preferred_element_type=jnp.float32)
             + jnp.dot(context.astype(jnp.bfloat16), wf2_ref[...],
                       preferred_element_type=jnp.float32) + bfuse_ref[...])
    out_ref[0] = x + read_gate * fused * scal_ref[_RSC]

    pmax = jnp.max(attn, axis=0, keepdims=True)
    @pl.when(t == 0)
    def _():
        smax_ref[...] = pmax
        best_ref[0] = -1.0
    @pl.when(t != 0)
    def _():
        smax_ref[...] = jnp.maximum(smax_ref[...], pmax)

    wm = jnp.dot(x, wto_ref[...]) + bto_ref[...]
    wm_ref[...] = wm
    il = jnp.dot(jax.nn.silu(
        jnp.dot(_ln(x, im_lng_ref[...], im_lnb_ref[...]), im_w1_ref[...])
        + im_b1_ref[...]), im_w2_ref[...])[:, :1] + scal_ref[_IMB2]
    imp = jax.nn.sigmoid(il) * jnp.abs(scal_ref[_WSC])
    cat = jnp.concatenate([q, read], axis=-1)
    wl = jnp.dot(jax.nn.silu(
        jnp.dot(_ln(cat, wd_lng_ref[...], wd_lnb_ref[...]), wd_w1_ref[...])
        + wd_b1_ref[...]), wd_w2_ref[...])[:, :1] + scal_ref[_WDB2]
    temp = jnp.maximum(scal_ref[_TMP], 0.1)
    adj = (wl + scal_ref[_WGB]) / temp
    ew = jnp.exp(jnp.minimum(adj, 10.0))
    strength = imp * (ew / (1.0 + ew))

    mt = jnp.max(strength)
    ridx = lax.broadcasted_iota(jnp.int32, (TS, 1), 0)
    it = jnp.min(jnp.where(strength >= mt, ridx, TS))

    @pl.when(mt > best_ref[0])
    def _():
        best_ref[0] = mt
        bestvec_ref[...] = wm_ref[pl.ds(it, 1), :]

    @pl.when(t == NT - 1)
    def _():
        v2 = valid * (1.0 - smax_ref[...])
        v2_ref[0] = v2
        mn = jnp.min(v2)
        cidx = lax.broadcasted_iota(jnp.int32, (1, N_SLOTS), 1)
        slot = jnp.min(jnp.where(v2 <= mn, cidx, N_SLOTS))
        slotg_ref[0] = jnp.broadcast_to(b * N_SLOTS + slot, (1, 1))
        bs = jnp.broadcast_to(jnp.clip(best_ref[0], 0.0, 1.0), (1, 1))
        slotvec_ref[0] = jnp.concatenate(
            [bestvec_ref[...], bs, temporal_ref[...],
             jnp.zeros((1, SV_PAD - D_SLOT), jnp.float32)], axis=-1)


def _tc_call(x, content, valid_row, scal, weights):
    (wq, bq, wto, bto, wfrom, bfrom, wf1, wf2, bfuse,
     rg_lng, rg_lnb, rg_w1, rg_b1, rg_w2,
     im_lng, im_lnb, im_w1, im_b1, im_w2,
     wd_lng, wd_lnb, wd_w1, wd_b1, wd_w2, temporal) = weights

    full = lambda shape: pl.BlockSpec(shape, lambda b, t: (0,) * len(shape))
    grid = (B, NT)
    specs = [
        pl.BlockSpec(memory_space=pltpu.SMEM),
        pl.BlockSpec((1, TS, D_MODEL), lambda b, t: (b, t, 0)),
        pl.BlockSpec((1, N_SLOTS, D_CACHE), lambda b, t: (b, 0, 0)),
        pl.BlockSpec((1, 1, N_SLOTS), lambda b, t: (b, 0, 0)),
    ] + [full(w.shape) for w in weights]
    out_shapes = [
        jax.ShapeDtypeStruct((B, S, D_MODEL), jnp.float32),
        jax.ShapeDtypeStruct((B, 1, N_SLOTS), jnp.float32),
        jax.ShapeDtypeStruct((B, 1, 1), jnp.int32),
        jax.ShapeDtypeStruct((B, 1, SV_PAD), jnp.float32),
    ]
    out_specs = [
        pl.BlockSpec((1, TS, D_MODEL), lambda b, t: (b, t, 0)),
        pl.BlockSpec((1, 1, N_SLOTS), lambda b, t: (b, 0, 0)),
        pl.BlockSpec((1, 1, 1), lambda b, t: (b, 0, 0)),
        pl.BlockSpec((1, 1, SV_PAD), lambda b, t: (b, 0, 0)),
    ]
    return pl.pallas_call(
        _tc_body,
        grid=grid,
        in_specs=specs,
        out_specs=out_specs,
        out_shape=out_shapes,
        scratch_shapes=[
            pltpu.VMEM((1, N_SLOTS), jnp.float32),
            pltpu.SMEM((1,), jnp.float32),
            pltpu.VMEM((1, D_CACHE), jnp.float32),
            pltpu.VMEM((TS, D_CACHE), jnp.float32),
        ],
        compiler_params=pltpu.CompilerParams(
            dimension_semantics=("arbitrary", "arbitrary")),
    )(scal, x, content, valid_row, *weights)


ROWS_PER_W = B * N_SLOTS // 32
WORDS_PER_W = ROWS_PER_W * D_SLOT


def _sc_body(mem_hbm, v2_hbm, slotg_hbm, slotvec_hbm, out_hbm,
             buf, v2v, svv, sgv):
    wid = lax.axis_index("s") * 2 + lax.axis_index("c")
    base_r = wid * ROWS_PER_W
    base_w = wid * WORDS_PER_W
    pltpu.sync_copy(mem_hbm.at[pl.ds(base_w, WORDS_PER_W)], buf)
    pltpu.sync_copy(v2_hbm.at[pl.ds(base_r, ROWS_PER_W)], v2v)
    pltpu.sync_copy(slotg_hbm, sgv)
    pltpu.sync_copy(slotvec_hbm, svv)
    lanes = lax.iota(jnp.int32, 16)
    for i in range(ROWS_PER_W // 16):
        idx = (i * 16 + lanes) * D_SLOT + D_CACHE
        plsc.store_scatter(buf, [idx], v2v[pl.ds(i * 16, 16)])
    sg = sgv[...]
    for bb in range(B):
        g = jnp.max(jnp.where(lanes == bb, sg, -1))
        local = g - base_r
        @pl.when((local >= 0) & (local < ROWS_PER_W))
        def _():
            for c in range(SV_PAD // 16):
                widx = local * D_SLOT + c * 16 + lanes
                mask = (c * 16 + lanes) < D_SLOT
                plsc.store_scatter(buf, [widx],
                                   svv[pl.ds(bb * SV_PAD + c * 16, 16)],
                                   mask=mask)
    pltpu.sync_copy(buf, out_hbm.at[pl.ds(base_w, WORDS_PER_W)])


@functools.cache
def _sc_update_kernel():
    return functools.partial(
        pl.kernel,
        mesh=plsc.VectorSubcoreMesh(core_axis_name="c", subcore_axis_name="s",
                                    num_cores=2),
        out_type=jax.ShapeDtypeStruct((B * N_SLOTS * D_SLOT,), jnp.float32),
        scratch_types=[
            pltpu.VMEM((WORDS_PER_W,), jnp.float32),
            pltpu.VMEM((ROWS_PER_W,), jnp.float32),
            pltpu.VMEM((B * SV_PAD,), jnp.float32),
            pltpu.VMEM((16,), jnp.int32),
        ],
        compiler_params=pltpu.CompilerParams(use_tc_tiling_on_sc=False,
                                             needs_layout_passes=False),
    )(_sc_body)


def kernel(x, memory, params):
    p = params
    content = memory[..., :D_CACHE]
    valid_row = memory[:, :, D_CACHE][:, None, :]

    scal = jnp.stack([
        p['read_gate_bias'], p['write_gate_bias'], p['read_scale'],
        p['write_scale'], p['write_temperature'], p['read_log_decay'],
        p['rg_b2'][0], p['im_b2'][0], p['wd_b2'][0],
    ]).astype(jnp.float32)

    row = lambda a: a.reshape(1, -1)
    weights = (
        p['Wq'], row(p['bq']), p['Wto'], row(p['bto']),
        p['Wfrom'], row(p['bfrom']),
        p['Wfuse'][:D_MODEL].astype(jnp.bfloat16),
        p['Wfuse'][D_MODEL:].astype(jnp.bfloat16), row(p['bfuse']),
        row(p['rg_lng']), row(p['rg_lnb']),
        p['rg_w1'].astype(jnp.bfloat16), row(p['rg_b1']),
        p['rg_w2'],
        row(p['im_lng']), row(p['im_lnb']), p['im_w1'], row(p['im_b1']),
        p['im_w2'],
        row(p['wd_lng']), row(p['wd_lnb']), p['wd_w1'], row(p['wd_b1']),
        p['wd_w2'],
        row(p['temporal_vec']),
    )

    out, v2, slotg, slotvec = _tc_call(x, content, valid_row, scal, weights)

    sg16 = jnp.zeros((16,), jnp.int32).at[:B].set(slotg.reshape(B))
    new_mem = _sc_update_kernel()(memory.reshape(-1), v2.reshape(-1), sg16,
                                  slotvec.reshape(-1))
    return out, new_mem.reshape(B, N_SLOTS, D_SLOT)


# speedup vs baseline: 1.6299x; 1.0670x over previous
import functools

import jax
import jax.numpy as jnp
from jax import lax
from jax.experimental import pallas as pl
from jax.experimental.pallas import tpu as pltpu
from jax.experimental.pallas import tpu_sc as plsc

B = 2
S = 2048
D_MODEL = 1024
D_CACHE = 64
N_SLOTS = 4096
D_TEMP = 16
D_SLOT = D_CACHE + 1 + D_TEMP

TS = 256
NT = S // TS

SV_PAD = 96

_RGB, _WGB, _RSC, _WSC, _TMP, _RLD, _RGB2, _IMB2, _WDB2 = range(9)


def _ln(x, g, b):
    m = x.mean(-1, keepdims=True)
    v = ((x - m) ** 2).mean(-1, keepdims=True)
    return (x - m) / jnp.sqrt(v + 1e-5) * g + b


def _tc_body(scal_ref, x_ref, content_ref, valid_ref,
             wq_ref, bq_ref, wto_ref, bto_ref, wfrom_ref, bfrom_ref,
             wf1_ref, wf2_ref, bfuse_ref,
             rg_lng_ref, rg_lnb_ref, rg_w1_ref, rg_b1_ref, rg_w2_ref,
             im_lng_ref, im_lnb_ref, im_w1_ref, im_b1_ref, im_w2_ref,
             wd_lng_ref, wd_lnb_ref, wd_w1_ref, wd_b1_ref, wd_w2_ref,
             temporal_ref,
             out_ref, v2_ref, sv_ref,
             smax_ref, best_ref, bestvec_ref, wm_ref):
    b = pl.program_id(0)
    t = pl.program_id(1)

    x = x_ref[0]
    content = content_ref[0]
    valid = valid_ref[0]

    q = jnp.dot(x, wq_ref[...]) + bq_ref[...]
    scores = lax.dot_general(q, content,
                             (((1,), (1,)), ((), ()))) / jnp.sqrt(float(D_CACHE))
    scores = jnp.where(valid > 0.5, scores, -1e9)
    e = jnp.exp(scores)
    attn = e * (1.0 / jnp.sum(e, axis=-1, keepdims=True))
    read = jnp.dot(attn, content)
    context = jnp.dot(read, wfrom_ref[...]) + bfrom_ref[...]

    gl = jnp.dot(jax.nn.silu(
        jnp.dot(_ln(x, rg_lng_ref[...], rg_lnb_ref[...]).astype(jnp.bfloat16),
                rg_w1_ref[...], preferred_element_type=jnp.float32)
        + rg_b1_ref[...]), rg_w2_ref[...])[:, :1] + scal_ref[_RGB2]
    r = jax.nn.sigmoid(gl + scal_ref[_RGB])
    base = jnp.clip(jax.nn.sigmoid(scal_ref[_RLD]), 0.1, 0.99)
    expo = jnp.clip(8.0 * r * jnp.log(base), -20.0, 0.0)
    read_gate = 1.0 - jnp.exp(expo)

    fused = (jnp.dot(x.astype(jnp.bfloat16), wf1_ref[...],
                     preferred_element_type=jnp.float32)
             + jnp.dot(context.astype(jnp.bfloat16), wf2_ref[...],
                       preferred_element_type=jnp.float32) + bfuse_ref[...])
    out_ref[0] = x + read_gate * fused * scal_ref[_RSC]

    pmax = jnp.max(attn, axis=0, keepdims=True)
    @pl.when(t == 0)
    def _():
        smax_ref[...] = pmax
        best_ref[0] = -1.0
    @pl.when(t != 0)
    def _():
        smax_ref[...] = jnp.maximum(smax_ref[...], pmax)

    wm = jnp.dot(x, wto_ref[...]) + bto_ref[...]
    wm_ref[...] = wm
    il = jnp.dot(jax.nn.silu(
        jnp.dot(_ln(x, im_lng_ref[...], im_lnb_ref[...]), im_w1_ref[...])
        + im_b1_ref[...]), im_w2_ref[...])[:, :1] + scal_ref[_IMB2]
    imp = jax.nn.sigmoid(il) * jnp.abs(scal_ref[_WSC])
    cat = jnp.concatenate([q, read], axis=-1)
    wl = jnp.dot(jax.nn.silu(
        jnp.dot(_ln(cat, wd_lng_ref[...], wd_lnb_ref[...]), wd_w1_ref[...])
        + wd_b1_ref[...]), wd_w2_ref[...])[:, :1] + scal_ref[_WDB2]
    temp = jnp.maximum(scal_ref[_TMP], 0.1)
    adj = (wl + scal_ref[_WGB]) / temp
    ew = jnp.exp(jnp.minimum(adj, 10.0))
    strength = imp * (ew / (1.0 + ew))

    mt = jnp.max(strength)
    ridx = lax.broadcasted_iota(jnp.int32, (TS, 1), 0)
    it = jnp.min(jnp.where(strength >= mt, ridx, TS))

    @pl.when(mt > best_ref[0])
    def _():
        best_ref[0] = mt
        bestvec_ref[...] = wm_ref[pl.ds(it, 1), :]

    @pl.when((b == 0) & (t == 0))
    def _():
        sv_ref[...] = jnp.zeros((16, 128), jnp.float32)

    @pl.when(t == NT - 1)
    def _():
        v2 = valid * (1.0 - smax_ref[...])
        v2_ref[0] = v2.reshape(8, N_SLOTS // 8)
        mn = jnp.min(v2)
        cidx = lax.broadcasted_iota(jnp.int32, (1, N_SLOTS), 1)
        slot = jnp.min(jnp.where(v2 <= mn, cidx, N_SLOTS))
        bs = jnp.broadcast_to(jnp.clip(best_ref[0], 0.0, 1.0), (1, 1))
        sg = jnp.broadcast_to((b * N_SLOTS + slot).astype(jnp.float32), (1, 1))
        sv_ref[pl.ds(b, 1), :] = jnp.concatenate(
            [bestvec_ref[...], bs, temporal_ref[...],
             jnp.zeros((1, 96 - D_SLOT), jnp.float32), sg,
             jnp.zeros((1, 31), jnp.float32)], axis=-1)


def _tc_call(x, content, valid_row, scal, weights):
    (wq, bq, wto, bto, wfrom, bfrom, wf1, wf2, bfuse,
     rg_lng, rg_lnb, rg_w1, rg_b1, rg_w2,
     im_lng, im_lnb, im_w1, im_b1, im_w2,
     wd_lng, wd_lnb, wd_w1, wd_b1, wd_w2, temporal) = weights

    full = lambda shape: pl.BlockSpec(shape, lambda b, t: (0,) * len(shape))
    grid = (B, NT)
    specs = [
        pl.BlockSpec(memory_space=pltpu.SMEM),
        pl.BlockSpec((1, TS, D_MODEL), lambda b, t: (b, t, 0)),
        pl.BlockSpec((1, N_SLOTS, D_CACHE), lambda b, t: (b, 0, 0)),
        pl.BlockSpec((1, 1, N_SLOTS), lambda b, t: (b, 0, 0)),
    ] + [full(w.shape) for w in weights]
    out_shapes = [
        jax.ShapeDtypeStruct((B, S, D_MODEL), jnp.float32),
        jax.ShapeDtypeStruct((B, 8, N_SLOTS // 8), jnp.float32),
        jax.ShapeDtypeStruct((16, 128), jnp.float32),
    ]
    out_specs = [
        pl.BlockSpec((1, TS, D_MODEL), lambda b, t: (b, t, 0)),
        pl.BlockSpec((1, 8, N_SLOTS // 8), lambda b, t: (b, 0, 0)),
        pl.BlockSpec((16, 128), lambda b, t: (0, 0)),
    ]
    return pl.pallas_call(
        _tc_body,
        grid=grid,
        in_specs=specs,
        out_specs=out_specs,
        out_shape=out_shapes,
        scratch_shapes=[
            pltpu.VMEM((1, N_SLOTS), jnp.float32),
            pltpu.SMEM((1,), jnp.float32),
            pltpu.VMEM((1, D_CACHE), jnp.float32),
            pltpu.VMEM((TS, D_CACHE), jnp.float32),
        ],
        compiler_params=pltpu.CompilerParams(
            dimension_semantics=("arbitrary", "arbitrary")),
    )(scal, x, content, valid_row, *weights)


ROWS_PER_W = B * N_SLOTS // 32
D_PAD = 128
WORDS_PER_W = ROWS_PER_W * D_PAD


def _sc_body(mem_hbm, v2_hbm, sv_hbm, out_hbm, buf, v2v, svv):
    wid = lax.axis_index("s") * 2 + lax.axis_index("c")
    base_r = wid * ROWS_PER_W
    base_w = wid * WORDS_PER_W
    pltpu.sync_copy(mem_hbm.at[pl.ds(base_w, WORDS_PER_W)], buf)
    pltpu.sync_copy(v2_hbm.at[pl.ds(base_r, ROWS_PER_W)], v2v)
    pltpu.sync_copy(sv_hbm, svv)
    lanes = lax.iota(jnp.int32, 16)
    for i in range(ROWS_PER_W // 16):
        idx = (i * 16 + lanes) * D_PAD + D_CACHE
        plsc.store_scatter(buf, [idx], v2v[pl.ds(i * 16, 16)])
    for bb in range(B):
        gv = svv[pl.ds(bb * 128 + 96, 16)]
        g = jnp.max(gv).astype(jnp.int32)
        local = g - base_r
        @pl.when((local >= 0) & (local < ROWS_PER_W))
        def _():
            for c in range(SV_PAD // 16):
                widx = local * D_PAD + c * 16 + lanes
                plsc.store_scatter(buf, [widx],
                                   svv[pl.ds(bb * 128 + c * 16, 16)])
    pltpu.sync_copy(buf, out_hbm.at[pl.ds(base_w, WORDS_PER_W)])


@functools.cache
def _sc_update_kernel():
    return functools.partial(
        pl.kernel,
        mesh=plsc.VectorSubcoreMesh(core_axis_name="c", subcore_axis_name="s",
                                    num_cores=2),
        out_type=jax.ShapeDtypeStruct((B * N_SLOTS * D_PAD,), jnp.float32),
        scratch_types=[
            pltpu.VMEM((WORDS_PER_W,), jnp.float32),
            pltpu.VMEM((ROWS_PER_W,), jnp.float32),
            pltpu.VMEM((16 * 128,), jnp.float32),
        ],
        compiler_params=pltpu.CompilerParams(use_tc_tiling_on_sc=False,
                                             needs_layout_passes=False),
    )(_sc_body)


def kernel(x, memory, params):
    p = params
    content = memory[..., :D_CACHE]
    valid_row = memory[:, :, D_CACHE][:, None, :]

    scal = jnp.stack([
        p['read_gate_bias'], p['write_gate_bias'], p['read_scale'],
        p['write_scale'], p['write_temperature'], p['read_log_decay'],
        p['rg_b2'][0], p['im_b2'][0], p['wd_b2'][0],
    ]).astype(jnp.float32)

    row = lambda a: a.reshape(1, -1)
    weights = (
        p['Wq'], row(p['bq']), p['Wto'], row(p['bto']),
        p['Wfrom'], row(p['bfrom']),
        p['Wfuse'][:D_MODEL].astype(jnp.bfloat16),
        p['Wfuse'][D_MODEL:].astype(jnp.bfloat16), row(p['bfuse']),
        row(p['rg_lng']), row(p['rg_lnb']),
        p['rg_w1'].astype(jnp.bfloat16), row(p['rg_b1']),
        p['rg_w2'],
        row(p['im_lng']), row(p['im_lnb']), p['im_w1'], row(p['im_b1']),
        p['im_w2'],
        row(p['wd_lng']), row(p['wd_lnb']), p['wd_w1'], row(p['wd_b1']),
        p['wd_w2'],
        row(p['temporal_vec']),
    )

    out, v2, sv = _tc_call(x, content, valid_row, scal, weights)

    mem_pad = jnp.pad(memory, ((0, 0), (0, 0), (0, D_PAD - D_SLOT)))
    new_mem = _sc_update_kernel()(mem_pad.reshape(-1), v2.reshape(-1),
                                  sv.reshape(-1))
    return out, new_mem.reshape(B, N_SLOTS, D_PAD)[..., :D_SLOT]


# speedup vs baseline: 1.7393x; 1.0671x over previous
import functools

import jax
import jax.numpy as jnp
from jax import lax
from jax.experimental import pallas as pl
from jax.experimental.pallas import tpu as pltpu
from jax.experimental.pallas import tpu_sc as plsc

B = 2
S = 2048
D_MODEL = 1024
D_CACHE = 64
N_SLOTS = 4096
D_TEMP = 16
D_SLOT = D_CACHE + 1 + D_TEMP

TS = 256
NT = S // TS

SV_PAD = 96
D_PAD = 128

_RGB, _WGB, _RSC, _WSC, _TMP, _RLD, _RGB2, _IMB2, _WDB2 = range(9)


def _ln(x, g, b):
    m = x.mean(-1, keepdims=True)
    v = ((x - m) ** 2).mean(-1, keepdims=True)
    return (x - m) / jnp.sqrt(v + 1e-5) * g + b


def _tc_body(scal_ref, x_ref, mem_ref, valid_ref,
             wq_ref, bq_ref, wto_ref, bto_ref, wfrom_ref, bfrom_ref,
             wfuse_ref, bfuse_ref,
             rg_lng_ref, rg_lnb_ref, rg_w1_ref, rg_b1_ref, rg_w2_ref,
             im_lng_ref, im_lnb_ref, im_w1_ref, im_b1_ref, im_w2_ref,
             wd_lng_ref, wd_lnb_ref, wd_w1_ref, wd_b1_ref, wd_w2_ref,
             temporal_ref,
             out_ref, v2_ref, sv_ref,
             smax_ref, best_ref, bestvec_ref, wm_ref):
    b = pl.program_id(0)
    t = pl.program_id(1)

    x = x_ref[0]
    content = mem_ref[0][:, :D_CACHE]
    valid = valid_ref[0]

    q = jnp.dot(x, wq_ref[...]) + bq_ref[...]
    scores = lax.dot_general(q, content,
                             (((1,), (1,)), ((), ()))) / jnp.sqrt(float(D_CACHE))
    scores = jnp.where(valid > 0.5, scores, -1e9)
    e = jnp.exp(scores)
    attn = e * (1.0 / jnp.sum(e, axis=-1, keepdims=True))
    read = jnp.dot(attn, content)
    context = jnp.dot(read, wfrom_ref[...]) + bfrom_ref[...]

    gl = jnp.dot(jax.nn.silu(
        jnp.dot(_ln(x, rg_lng_ref[...], rg_lnb_ref[...]).astype(jnp.bfloat16),
                rg_w1_ref[...], preferred_element_type=jnp.float32)
        + rg_b1_ref[...]), rg_w2_ref[...])[:, :1] + scal_ref[_RGB2]
    r = jax.nn.sigmoid(gl + scal_ref[_RGB])
    base = jnp.clip(jax.nn.sigmoid(scal_ref[_RLD]), 0.1, 0.99)
    expo = jnp.clip(8.0 * r * jnp.log(base), -20.0, 0.0)
    read_gate = 1.0 - jnp.exp(expo)

    xc = jnp.concatenate([x.astype(jnp.bfloat16),
                          context.astype(jnp.bfloat16)], axis=-1)
    fused = jnp.dot(xc, wfuse_ref[...],
                    preferred_element_type=jnp.float32) + bfuse_ref[...]
    out_ref[0] = x + read_gate * fused * scal_ref[_RSC]

    pmax = jnp.max(attn, axis=0, keepdims=True)
    @pl.when(t == 0)
    def _():
        smax_ref[...] = pmax
        best_ref[0] = -1.0
    @pl.when(t != 0)
    def _():
        smax_ref[...] = jnp.maximum(smax_ref[...], pmax)

    wm = jnp.dot(x, wto_ref[...]) + bto_ref[...]
    wm_ref[...] = wm
    il = jnp.dot(jax.nn.silu(
        jnp.dot(_ln(x, im_lng_ref[...], im_lnb_ref[...]), im_w1_ref[...])
        + im_b1_ref[...]), im_w2_ref[...])[:, :1] + scal_ref[_IMB2]
    imp = jax.nn.sigmoid(il) * jnp.abs(scal_ref[_WSC])
    cat = jnp.concatenate([q, read], axis=-1)
    wl = jnp.dot(jax.nn.silu(
        jnp.dot(_ln(cat, wd_lng_ref[...], wd_lnb_ref[...]), wd_w1_ref[...])
        + wd_b1_ref[...]), wd_w2_ref[...])[:, :1] + scal_ref[_WDB2]
    temp = jnp.maximum(scal_ref[_TMP], 0.1)
    adj = (wl + scal_ref[_WGB]) / temp
    ew = jnp.exp(jnp.minimum(adj, 10.0))
    strength = imp * (ew / (1.0 + ew))

    mt = jnp.max(strength)
    ridx = lax.broadcasted_iota(jnp.int32, (TS, 1), 0)
    it = jnp.min(jnp.where(strength >= mt, ridx, TS))

    @pl.when(mt > best_ref[0])
    def _():
        best_ref[0] = mt
        bestvec_ref[...] = wm_ref[pl.ds(it, 1), :]

    @pl.when((b == 0) & (t == 0))
    def _():
        sv_ref[...] = jnp.zeros((16, 128), jnp.float32)

    @pl.when(t == NT - 1)
    def _():
        v2 = valid * (1.0 - smax_ref[...])
        v2_ref[0] = v2.reshape(8, N_SLOTS // 8)
        mn = jnp.min(v2)
        cidx = lax.broadcasted_iota(jnp.int32, (1, N_SLOTS), 1)
        slot = jnp.min(jnp.where(v2 <= mn, cidx, N_SLOTS))
        bs = jnp.broadcast_to(jnp.clip(best_ref[0], 0.0, 1.0), (1, 1))
        sg = jnp.broadcast_to((b * N_SLOTS + slot).astype(jnp.float32), (1, 1))
        sv_ref[pl.ds(b, 1), :] = jnp.concatenate(
            [bestvec_ref[...], bs, temporal_ref[...],
             jnp.zeros((1, 96 - D_SLOT), jnp.float32), sg,
             jnp.zeros((1, 31), jnp.float32)], axis=-1)


def _tc_call(x, mem_pad, valid_row, scal, weights):
    (wq, bq, wto, bto, wfrom, bfrom, wfuse, bfuse,
     rg_lng, rg_lnb, rg_w1, rg_b1, rg_w2,
     im_lng, im_lnb, im_w1, im_b1, im_w2,
     wd_lng, wd_lnb, wd_w1, wd_b1, wd_w2, temporal) = weights

    full = lambda shape: pl.BlockSpec(shape, lambda b, t: (0,) * len(shape))
    grid = (B, NT)
    specs = [
        pl.BlockSpec(memory_space=pltpu.SMEM),
        pl.BlockSpec((1, TS, D_MODEL), lambda b, t: (b, t, 0)),
        pl.BlockSpec((1, N_SLOTS, D_PAD), lambda b, t: (b, 0, 0)),
        pl.BlockSpec((1, 1, N_SLOTS), lambda b, t: (b, 0, 0)),
    ] + [full(w.shape) for w in weights]
    out_shapes = [
        jax.ShapeDtypeStruct((B, S, D_MODEL), jnp.float32),
        jax.ShapeDtypeStruct((B, 8, N_SLOTS // 8), jnp.float32),
        jax.ShapeDtypeStruct((16, 128), jnp.float32),
    ]
    out_specs = [
        pl.BlockSpec((1, TS, D_MODEL), lambda b, t: (b, t, 0)),
        pl.BlockSpec((1, 8, N_SLOTS // 8), lambda b, t: (b, 0, 0)),
        pl.BlockSpec((16, 128), lambda b, t: (0, 0)),
    ]
    return pl.pallas_call(
        _tc_body,
        grid=grid,
        in_specs=specs,
        out_specs=out_specs,
        out_shape=out_shapes,
        scratch_shapes=[
            pltpu.VMEM((1, N_SLOTS), jnp.float32),
            pltpu.SMEM((1,), jnp.float32),
            pltpu.VMEM((1, D_CACHE), jnp.float32),
            pltpu.VMEM((TS, D_CACHE), jnp.float32),
        ],
        compiler_params=pltpu.CompilerParams(
            dimension_semantics=("arbitrary", "arbitrary")),
    )(scal, x, mem_pad, valid_row, *weights)


ROWS_PER_W = B * N_SLOTS // 32
WORDS_PER_W = ROWS_PER_W * D_PAD


def _sc_body(mem_hbm, v2_hbm, sv_hbm, out_hbm, buf, v2v, svv):
    wid = lax.axis_index("s") * 2 + lax.axis_index("c")
    base_r = wid * ROWS_PER_W
    base_w = wid * WORDS_PER_W
    pltpu.sync_copy(mem_hbm.at[pl.ds(base_w, WORDS_PER_W)], buf)
    pltpu.sync_copy(v2_hbm.at[pl.ds(base_r, ROWS_PER_W)], v2v)
    pltpu.sync_copy(sv_hbm, svv)
    lanes = lax.iota(jnp.int32, 16)
    for i in range(ROWS_PER_W // 16):
        idx = (i * 16 + lanes) * D_PAD + D_CACHE
        plsc.store_scatter(buf, [idx], v2v[pl.ds(i * 16, 16)])
    for bb in range(B):
        gv = svv[pl.ds(bb * 128 + 96, 16)]
        g = jnp.max(gv).astype(jnp.int32)
        local = g - base_r
        @pl.when((local >= 0) & (local < ROWS_PER_W))
        def _():
            for c in range(SV_PAD // 16):
                widx = local * D_PAD + c * 16 + lanes
                plsc.store_scatter(buf, [widx],
                                   svv[pl.ds(bb * 128 + c * 16, 16)])
    pltpu.sync_copy(buf, out_hbm.at[pl.ds(base_w, WORDS_PER_W)])


@functools.cache
def _sc_update_kernel():
    return functools.partial(
        pl.kernel,
        mesh=plsc.VectorSubcoreMesh(core_axis_name="c", subcore_axis_name="s",
                                    num_cores=2),
        out_type=jax.ShapeDtypeStruct((B * N_SLOTS * D_PAD,), jnp.float32),
        scratch_types=[
            pltpu.VMEM((WORDS_PER_W,), jnp.float32),
            pltpu.VMEM((ROWS_PER_W,), jnp.float32),
            pltpu.VMEM((16 * 128,), jnp.float32),
        ],
        compiler_params=pltpu.CompilerParams(use_tc_tiling_on_sc=False,
                                             needs_layout_passes=False),
    )(_sc_body)


def kernel(x, memory, params):
    p = params
    mem_pad = jnp.pad(memory, ((0, 0), (0, 0), (0, D_PAD - D_SLOT)))
    valid_row = memory[:, :, D_CACHE][:, None, :]

    scal = jnp.stack([
        p['read_gate_bias'], p['write_gate_bias'], p['read_scale'],
        p['write_scale'], p['write_temperature'], p['read_log_decay'],
        p['rg_b2'][0], p['im_b2'][0], p['wd_b2'][0],
    ]).astype(jnp.float32)

    row = lambda a: a.reshape(1, -1)
    weights = (
        p['Wq'], row(p['bq']), p['Wto'], row(p['bto']),
        p['Wfrom'], row(p['bfrom']),
        p['Wfuse'].astype(jnp.bfloat16), row(p['bfuse']),
        row(p['rg_lng']), row(p['rg_lnb']),
        p['rg_w1'].astype(jnp.bfloat16), row(p['rg_b1']),
        p['rg_w2'],
        row(p['im_lng']), row(p['im_lnb']), p['im_w1'], row(p['im_b1']),
        p['im_w2'],
        row(p['wd_lng']), row(p['wd_lnb']), p['wd_w1'], row(p['wd_b1']),
        p['wd_w2'],
        row(p['temporal_vec']),
    )

    out, v2, sv = _tc_call(x, mem_pad, valid_row, scal, weights)

    new_mem = _sc_update_kernel()(mem_pad.reshape(-1), v2.reshape(-1),
                                  sv.reshape(-1))
    return out, new_mem.reshape(B, N_SLOTS, D_PAD)[..., :D_SLOT]


# speedup vs baseline: 1.8711x; 1.0758x over previous
import functools

import jax
import jax.numpy as jnp
from jax import lax
from jax.experimental import pallas as pl
from jax.experimental.pallas import tpu as pltpu
from jax.experimental.pallas import tpu_sc as plsc

B = 2
S = 2048
D_MODEL = 1024
D_CACHE = 64
N_SLOTS = 4096
D_TEMP = 16
D_SLOT = D_CACHE + 1 + D_TEMP

TS = 512
NT = S // TS

SV_PAD = 96
D_PAD = 128

_RGB, _WGB, _RSC, _WSC, _TMP, _RLD, _RGB2, _IMB2, _WDB2 = range(9)


def _ln(x, g, b):
    m = x.mean(-1, keepdims=True)
    v = ((x - m) ** 2).mean(-1, keepdims=True)
    return (x - m) / jnp.sqrt(v + 1e-5) * g + b


def _tc_body(scal_ref, x_ref, mem_ref, valid_ref,
             wq_ref, bq_ref, wto_ref, bto_ref, wfrom_ref, bfrom_ref,
             wfuse_ref, bfuse_ref,
             rg_lng_ref, rg_lnb_ref, rg_w1_ref, rg_b1_ref, rg_w2_ref,
             im_lng_ref, im_lnb_ref, im_w1_ref, im_b1_ref, im_w2_ref,
             wd_lng_ref, wd_lnb_ref, wd_w1_ref, wd_b1_ref, wd_w2_ref,
             temporal_ref,
             out_ref, v2_ref, sv_ref,
             smax_ref, best_ref, bestvec_ref, wm_ref):
    b = pl.program_id(0)
    t = pl.program_id(1)

    x = x_ref[0]
    content = mem_ref[0][:, :D_CACHE]
    valid = valid_ref[0]

    q = jnp.dot(x, wq_ref[...]) + bq_ref[...]
    scores = lax.dot_general(q, content,
                             (((1,), (1,)), ((), ()))) / jnp.sqrt(float(D_CACHE))
    scores = jnp.where(valid > 0.5, scores, -1e9)
    e = jnp.exp(scores)
    attn = e * (1.0 / jnp.sum(e, axis=-1, keepdims=True))
    read = jnp.dot(attn, content)
    context = jnp.dot(read, wfrom_ref[...]) + bfrom_ref[...]

    gl = jnp.dot(jax.nn.silu(
        jnp.dot(_ln(x, rg_lng_ref[...], rg_lnb_ref[...]).astype(jnp.bfloat16),
                rg_w1_ref[...], preferred_element_type=jnp.float32)
        + rg_b1_ref[...]), rg_w2_ref[...])[:, :1] + scal_ref[_RGB2]
    r = jax.nn.sigmoid(gl + scal_ref[_RGB])
    base = jnp.clip(jax.nn.sigmoid(scal_ref[_RLD]), 0.1, 0.99)
    expo = jnp.clip(8.0 * r * jnp.log(base), -20.0, 0.0)
    read_gate = 1.0 - jnp.exp(expo)

    xc = jnp.concatenate([x.astype(jnp.bfloat16),
                          context.astype(jnp.bfloat16)], axis=-1)
    fused = jnp.dot(xc, wfuse_ref[...],
                    preferred_element_type=jnp.float32) + bfuse_ref[...]
    out_ref[0] = x + read_gate * fused * scal_ref[_RSC]

    pmax = jnp.max(attn, axis=0, keepdims=True)
    @pl.when(t == 0)
    def _():
        smax_ref[...] = pmax
        best_ref[0] = -1.0
    @pl.when(t != 0)
    def _():
        smax_ref[...] = jnp.maximum(smax_ref[...], pmax)

    wm = jnp.dot(x, wto_ref[...]) + bto_ref[...]
    wm_ref[...] = wm
    il = jnp.dot(jax.nn.silu(
        jnp.dot(_ln(x, im_lng_ref[...], im_lnb_ref[...]), im_w1_ref[...])
        + im_b1_ref[...]), im_w2_ref[...])[:, :1] + scal_ref[_IMB2]
    imp = jax.nn.sigmoid(il) * jnp.abs(scal_ref[_WSC])
    cat = jnp.concatenate([q, read], axis=-1)
    wl = jnp.dot(jax.nn.silu(
        jnp.dot(_ln(cat, wd_lng_ref[...], wd_lnb_ref[...]), wd_w1_ref[...])
        + wd_b1_ref[...]), wd_w2_ref[...])[:, :1] + scal_ref[_WDB2]
    temp = jnp.maximum(scal_ref[_TMP], 0.1)
    adj = (wl + scal_ref[_WGB]) / temp
    ew = jnp.exp(jnp.minimum(adj, 10.0))
    strength = imp * (ew / (1.0 + ew))

    mt = jnp.max(strength)
    ridx = lax.broadcasted_iota(jnp.int32, (TS, 1), 0)
    it = jnp.min(jnp.where(strength >= mt, ridx, TS))

    @pl.when(mt > best_ref[0])
    def _():
        best_ref[0] = mt
        bestvec_ref[...] = wm_ref[pl.ds(it, 1), :]

    @pl.when((b == 0) & (t == 0))
    def _():
        sv_ref[...] = jnp.zeros((16, 128), jnp.float32)

    @pl.when(t == NT - 1)
    def _():
        v2 = valid * (1.0 - smax_ref[...])
        v2_ref[0] = v2.reshape(8, N_SLOTS // 8)
        mn = jnp.min(v2)
        cidx = lax.broadcasted_iota(jnp.int32, (1, N_SLOTS), 1)
        slot = jnp.min(jnp.where(v2 <= mn, cidx, N_SLOTS))
        bs = jnp.broadcast_to(jnp.clip(best_ref[0], 0.0, 1.0), (1, 1))
        sg = jnp.broadcast_to((b * N_SLOTS + slot).astype(jnp.float32), (1, 1))
        sv_ref[pl.ds(b, 1), :] = jnp.concatenate(
            [bestvec_ref[...], bs, temporal_ref[...],
             jnp.zeros((1, 96 - D_SLOT), jnp.float32), sg,
             jnp.zeros((1, 31), jnp.float32)], axis=-1)


def _tc_call(x, mem_pad, valid_row, scal, weights):
    (wq, bq, wto, bto, wfrom, bfrom, wfuse, bfuse,
     rg_lng, rg_lnb, rg_w1, rg_b1, rg_w2,
     im_lng, im_lnb, im_w1, im_b1, im_w2,
     wd_lng, wd_lnb, wd_w1, wd_b1, wd_w2, temporal) = weights

    full = lambda shape: pl.BlockSpec(shape, lambda b, t: (0,) * len(shape))
    grid = (B, NT)
    specs = [
        pl.BlockSpec(memory_space=pltpu.SMEM),
        pl.BlockSpec((1, TS, D_MODEL), lambda b, t: (b, t, 0)),
        pl.BlockSpec((1, N_SLOTS, D_PAD), lambda b, t: (b, 0, 0)),
        pl.BlockSpec((1, 1, N_SLOTS), lambda b, t: (b, 0, 0)),
    ] + [full(w.shape) for w in weights]
    out_shapes = [
        jax.ShapeDtypeStruct((B, S, D_MODEL), jnp.float32),
        jax.ShapeDtypeStruct((B, 8, N_SLOTS // 8), jnp.float32),
        jax.ShapeDtypeStruct((16, 128), jnp.float32),
    ]
    out_specs = [
        pl.BlockSpec((1, TS, D_MODEL), lambda b, t: (b, t, 0)),
        pl.BlockSpec((1, 8, N_SLOTS // 8), lambda b, t: (b, 0, 0)),
        pl.BlockSpec((16, 128), lambda b, t: (0, 0)),
    ]
    return pl.pallas_call(
        _tc_body,
        grid=grid,
        in_specs=specs,
        out_specs=out_specs,
        out_shape=out_shapes,
        scratch_shapes=[
            pltpu.VMEM((1, N_SLOTS), jnp.float32),
            pltpu.SMEM((1,), jnp.float32),
            pltpu.VMEM((1, D_CACHE), jnp.float32),
            pltpu.VMEM((TS, D_CACHE), jnp.float32),
        ],
        compiler_params=pltpu.CompilerParams(
            dimension_semantics=("arbitrary", "arbitrary")),
    )(scal, x, mem_pad, valid_row, *weights)


ROWS_PER_W = B * N_SLOTS // 32
WORDS_PER_W = ROWS_PER_W * D_PAD


def _sc_body(mem_hbm, v2_hbm, sv_hbm, out_hbm, buf, v2v, svv):
    wid = lax.axis_index("s") * 2 + lax.axis_index("c")
    base_r = wid * ROWS_PER_W
    base_w = wid * WORDS_PER_W
    pltpu.sync_copy(mem_hbm.at[pl.ds(base_w, WORDS_PER_W)], buf)
    pltpu.sync_copy(v2_hbm.at[pl.ds(base_r, ROWS_PER_W)], v2v)
    pltpu.sync_copy(sv_hbm, svv)
    lanes = lax.iota(jnp.int32, 16)
    for i in range(ROWS_PER_W // 16):
        idx = (i * 16 + lanes) * D_PAD + D_CACHE
        plsc.store_scatter(buf, [idx], v2v[pl.ds(i * 16, 16)])
    for bb in range(B):
        gv = svv[pl.ds(bb * 128 + 96, 16)]
        g = jnp.max(gv).astype(jnp.int32)
        local = g - base_r
        @pl.when((local >= 0) & (local < ROWS_PER_W))
        def _():
            for c in range(SV_PAD // 16):
                widx = local * D_PAD + c * 16 + lanes
                plsc.store_scatter(buf, [widx],
                                   svv[pl.ds(bb * 128 + c * 16, 16)])
    pltpu.sync_copy(buf, out_hbm.at[pl.ds(base_w, WORDS_PER_W)])


@functools.cache
def _sc_update_kernel():
    return functools.partial(
        pl.kernel,
        mesh=plsc.VectorSubcoreMesh(core_axis_name="c", subcore_axis_name="s",
                                    num_cores=2),
        out_type=jax.ShapeDtypeStruct((B * N_SLOTS * D_PAD,), jnp.float32),
        scratch_types=[
            pltpu.VMEM((WORDS_PER_W,), jnp.float32),
            pltpu.VMEM((ROWS_PER_W,), jnp.float32),
            pltpu.VMEM((16 * 128,), jnp.float32),
        ],
        compiler_params=pltpu.CompilerParams(use_tc_tiling_on_sc=False,
                                             needs_layout_passes=False),
    )(_sc_body)


def kernel(x, memory, params):
    p = params
    mem_pad = jnp.pad(memory, ((0, 0), (0, 0), (0, D_PAD - D_SLOT)))
    valid_row = memory[:, :, D_CACHE][:, None, :]

    scal = jnp.stack([
        p['read_gate_bias'], p['write_gate_bias'], p['read_scale'],
        p['write_scale'], p['write_temperature'], p['read_log_decay'],
        p['rg_b2'][0], p['im_b2'][0], p['wd_b2'][0],
    ]).astype(jnp.float32)

    row = lambda a: a.reshape(1, -1)
    weights = (
        p['Wq'], row(p['bq']), p['Wto'], row(p['bto']),
        p['Wfrom'], row(p['bfrom']),
        p['Wfuse'].astype(jnp.bfloat16), row(p['bfuse']),
        row(p['rg_lng']), row(p['rg_lnb']),
        p['rg_w1'].astype(jnp.bfloat16), row(p['rg_b1']),
        p['rg_w2'],
        row(p['im_lng']), row(p['im_lnb']), p['im_w1'], row(p['im_b1']),
        p['im_w2'],
        row(p['wd_lng']), row(p['wd_lnb']), p['wd_w1'], row(p['wd_b1']),
        p['wd_w2'],
        row(p['temporal_vec']),
    )

    out, v2, sv = _tc_call(x, mem_pad, valid_row, scal, weights)

    new_mem = _sc_update_kernel()(mem_pad.reshape(-1), v2.reshape(-1),
                                  sv.reshape(-1))
    return out, new_mem.reshape(B, N_SLOTS, D_PAD)[..., :D_SLOT]


# speedup vs baseline: 1.9004x; 1.0156x over previous
import functools

import jax
import jax.numpy as jnp
from jax import lax
from jax.experimental import pallas as pl
from jax.experimental.pallas import tpu as pltpu
from jax.experimental.pallas import tpu_sc as plsc

B = 2
S = 2048
D_MODEL = 1024
D_CACHE = 64
N_SLOTS = 4096
D_TEMP = 16
D_SLOT = D_CACHE + 1 + D_TEMP

TS = 1024
NT = S // TS

SV_PAD = 96
D_PAD = 128

_RGB, _WGB, _RSC, _WSC, _TMP, _RLD, _RGB2, _IMB2, _WDB2 = range(9)


def _ln(x, g, b):
    m = x.mean(-1, keepdims=True)
    v = ((x - m) ** 2).mean(-1, keepdims=True)
    return (x - m) / jnp.sqrt(v + 1e-5) * g + b


def _tc_body(scal_ref, x_ref, mem_ref, valid_ref,
             wq_ref, bq_ref, wto_ref, bto_ref, wfrom_ref, bfrom_ref,
             wfuse_ref, bfuse_ref,
             rg_lng_ref, rg_lnb_ref, rg_w1_ref, rg_b1_ref, rg_w2_ref,
             im_lng_ref, im_lnb_ref, im_w1_ref, im_b1_ref, im_w2_ref,
             wd_lng_ref, wd_lnb_ref, wd_w1_ref, wd_b1_ref, wd_w2_ref,
             temporal_ref,
             out_ref, v2_ref, sv_ref,
             smax_ref, best_ref, bestvec_ref, wm_ref):
    b = pl.program_id(0)
    t = pl.program_id(1)

    x = x_ref[0]
    content = mem_ref[0][:, :D_CACHE]
    valid = valid_ref[0]

    q = jnp.dot(x, wq_ref[...]) + bq_ref[...]
    scores = lax.dot_general(q, content,
                             (((1,), (1,)), ((), ()))) / jnp.sqrt(float(D_CACHE))
    scores = jnp.where(valid > 0.5, scores, -1e9)
    e = jnp.exp(scores)
    attn = e * (1.0 / jnp.sum(e, axis=-1, keepdims=True))
    read = jnp.dot(attn, content)
    context = jnp.dot(read, wfrom_ref[...]) + bfrom_ref[...]

    gl = jnp.dot(jax.nn.silu(
        jnp.dot(_ln(x, rg_lng_ref[...], rg_lnb_ref[...]).astype(jnp.bfloat16),
                rg_w1_ref[...], preferred_element_type=jnp.float32)
        + rg_b1_ref[...]), rg_w2_ref[...])[:, :1] + scal_ref[_RGB2]
    r = jax.nn.sigmoid(gl + scal_ref[_RGB])
    base = jnp.clip(jax.nn.sigmoid(scal_ref[_RLD]), 0.1, 0.99)
    expo = jnp.clip(8.0 * r * jnp.log(base), -20.0, 0.0)
    read_gate = 1.0 - jnp.exp(expo)

    xc = jnp.concatenate([x.astype(jnp.bfloat16),
                          context.astype(jnp.bfloat16)], axis=-1)
    fused = jnp.dot(xc, wfuse_ref[...],
                    preferred_element_type=jnp.float32) + bfuse_ref[...]
    out_ref[0] = x + read_gate * fused * scal_ref[_RSC]

    pmax = jnp.max(attn, axis=0, keepdims=True)
    @pl.when(t == 0)
    def _():
        smax_ref[...] = pmax
        best_ref[0] = -1.0
    @pl.when(t != 0)
    def _():
        smax_ref[...] = jnp.maximum(smax_ref[...], pmax)

    wm = jnp.dot(x, wto_ref[...]) + bto_ref[...]
    wm_ref[...] = wm
    il = jnp.dot(jax.nn.silu(
        jnp.dot(_ln(x, im_lng_ref[...], im_lnb_ref[...]), im_w1_ref[...])
        + im_b1_ref[...]), im_w2_ref[...])[:, :1] + scal_ref[_IMB2]
    imp = jax.nn.sigmoid(il) * jnp.abs(scal_ref[_WSC])
    cat = jnp.concatenate([q, read], axis=-1)
    wl = jnp.dot(jax.nn.silu(
        jnp.dot(_ln(cat, wd_lng_ref[...], wd_lnb_ref[...]), wd_w1_ref[...])
        + wd_b1_ref[...]), wd_w2_ref[...])[:, :1] + scal_ref[_WDB2]
    temp = jnp.maximum(scal_ref[_TMP], 0.1)
    adj = (wl + scal_ref[_WGB]) / temp
    ew = jnp.exp(jnp.minimum(adj, 10.0))
    strength = imp * (ew / (1.0 + ew))

    mt = jnp.max(strength)
    ridx = lax.broadcasted_iota(jnp.int32, (TS, 1), 0)
    it = jnp.min(jnp.where(strength >= mt, ridx, TS))

    @pl.when(mt > best_ref[0])
    def _():
        best_ref[0] = mt
        bestvec_ref[...] = wm_ref[pl.ds(it, 1), :]

    @pl.when((b == 0) & (t == 0))
    def _():
        sv_ref[...] = jnp.zeros((16, 128), jnp.float32)

    @pl.when(t == NT - 1)
    def _():
        v2 = valid * (1.0 - smax_ref[...])
        v2_ref[0] = v2.reshape(8, N_SLOTS // 8)
        mn = jnp.min(v2)
        cidx = lax.broadcasted_iota(jnp.int32, (1, N_SLOTS), 1)
        slot = jnp.min(jnp.where(v2 <= mn, cidx, N_SLOTS))
        bs = jnp.broadcast_to(jnp.clip(best_ref[0], 0.0, 1.0), (1, 1))
        sg = jnp.broadcast_to((b * N_SLOTS + slot).astype(jnp.float32), (1, 1))
        sv_ref[pl.ds(b, 1), :] = jnp.concatenate(
            [bestvec_ref[...], bs, temporal_ref[...],
             jnp.zeros((1, 96 - D_SLOT), jnp.float32), sg,
             jnp.zeros((1, 31), jnp.float32)], axis=-1)


def _tc_call(x, mem_pad, valid_row, scal, weights):
    (wq, bq, wto, bto, wfrom, bfrom, wfuse, bfuse,
     rg_lng, rg_lnb, rg_w1, rg_b1, rg_w2,
     im_lng, im_lnb, im_w1, im_b1, im_w2,
     wd_lng, wd_lnb, wd_w1, wd_b1, wd_w2, temporal) = weights

    full = lambda shape: pl.BlockSpec(shape, lambda b, t: (0,) * len(shape))
    grid = (B, NT)
    specs = [
        pl.BlockSpec(memory_space=pltpu.SMEM),
        pl.BlockSpec((1, TS, D_MODEL), lambda b, t: (b, t, 0)),
        pl.BlockSpec((1, N_SLOTS, D_PAD), lambda b, t: (b, 0, 0)),
        pl.BlockSpec((1, 1, N_SLOTS), lambda b, t: (b, 0, 0)),
    ] + [full(w.shape) for w in weights]
    out_shapes = [
        jax.ShapeDtypeStruct((B, S, D_MODEL), jnp.float32),
        jax.ShapeDtypeStruct((B, 8, N_SLOTS // 8), jnp.float32),
        jax.ShapeDtypeStruct((16, 128), jnp.float32),
    ]
    out_specs = [
        pl.BlockSpec((1, TS, D_MODEL), lambda b, t: (b, t, 0)),
        pl.BlockSpec((1, 8, N_SLOTS // 8), lambda b, t: (b, 0, 0)),
        pl.BlockSpec((16, 128), lambda b, t: (0, 0)),
    ]
    return pl.pallas_call(
        _tc_body,
        grid=grid,
        in_specs=specs,
        out_specs=out_specs,
        out_shape=out_shapes,
        scratch_shapes=[
            pltpu.VMEM((1, N_SLOTS), jnp.float32),
            pltpu.SMEM((1,), jnp.float32),
            pltpu.VMEM((1, D_CACHE), jnp.float32),
            pltpu.VMEM((TS, D_CACHE), jnp.float32),
        ],
        compiler_params=pltpu.CompilerParams(
            dimension_semantics=("arbitrary", "arbitrary")),
    )(scal, x, mem_pad, valid_row, *weights)


ROWS_PER_W = B * N_SLOTS // 32
WORDS_PER_W = ROWS_PER_W * D_PAD


def _sc_body(mem_hbm, v2_hbm, sv_hbm, out_hbm, buf, v2v, svv):
    wid = lax.axis_index("s") * 2 + lax.axis_index("c")
    base_r = wid * ROWS_PER_W
    base_w = wid * WORDS_PER_W
    pltpu.sync_copy(mem_hbm.at[pl.ds(base_w, WORDS_PER_W)], buf)
    pltpu.sync_copy(v2_hbm.at[pl.ds(base_r, ROWS_PER_W)], v2v)
    pltpu.sync_copy(sv_hbm, svv)
    lanes = lax.iota(jnp.int32, 16)
    for i in range(ROWS_PER_W // 16):
        idx = (i * 16 + lanes) * D_PAD + D_CACHE
        plsc.store_scatter(buf, [idx], v2v[pl.ds(i * 16, 16)])
    for bb in range(B):
        gv = svv[pl.ds(bb * 128 + 96, 16)]
        g = jnp.max(gv).astype(jnp.int32)
        local = g - base_r
        @pl.when((local >= 0) & (local < ROWS_PER_W))
        def _():
            for c in range(SV_PAD // 16):
                widx = local * D_PAD + c * 16 + lanes
                plsc.store_scatter(buf, [widx],
                                   svv[pl.ds(bb * 128 + c * 16, 16)])
    pltpu.sync_copy(buf, out_hbm.at[pl.ds(base_w, WORDS_PER_W)])


@functools.cache
def _sc_update_kernel():
    return functools.partial(
        pl.kernel,
        mesh=plsc.VectorSubcoreMesh(core_axis_name="c", subcore_axis_name="s",
                                    num_cores=2),
        out_type=jax.ShapeDtypeStruct((B * N_SLOTS * D_PAD,), jnp.float32),
        scratch_types=[
            pltpu.VMEM((WORDS_PER_W,), jnp.float32),
            pltpu.VMEM((ROWS_PER_W,), jnp.float32),
            pltpu.VMEM((16 * 128,), jnp.float32),
        ],
        compiler_params=pltpu.CompilerParams(use_tc_tiling_on_sc=False,
                                             needs_layout_passes=False),
    )(_sc_body)


def kernel(x, memory, params):
    p = params
    mem_pad = jnp.pad(memory, ((0, 0), (0, 0), (0, D_PAD - D_SLOT)))
    valid_row = memory[:, :, D_CACHE][:, None, :]

    scal = jnp.stack([
        p['read_gate_bias'], p['write_gate_bias'], p['read_scale'],
        p['write_scale'], p['write_temperature'], p['read_log_decay'],
        p['rg_b2'][0], p['im_b2'][0], p['wd_b2'][0],
    ]).astype(jnp.float32)

    row = lambda a: a.reshape(1, -1)
    weights = (
        p['Wq'], row(p['bq']), p['Wto'], row(p['bto']),
        p['Wfrom'], row(p['bfrom']),
        p['Wfuse'].astype(jnp.bfloat16), row(p['bfuse']),
        row(p['rg_lng']), row(p['rg_lnb']),
        p['rg_w1'].astype(jnp.bfloat16), row(p['rg_b1']),
        p['rg_w2'],
        row(p['im_lng']), row(p['im_lnb']), p['im_w1'], row(p['im_b1']),
        p['im_w2'],
        row(p['wd_lng']), row(p['wd_lnb']), p['wd_w1'], row(p['wd_b1']),
        p['wd_w2'],
        row(p['temporal_vec']),
    )

    out, v2, sv = _tc_call(x, mem_pad, valid_row, scal, weights)

    new_mem = _sc_update_kernel()(mem_pad.reshape(-1), v2.reshape(-1),
                                  sv.reshape(-1))
    return out, new_mem.reshape(B, N_SLOTS, D_PAD)[..., :D_SLOT]
